# baseline (speedup 1.0000x reference)
import functools

import jax
import jax.numpy as jnp
from jax.experimental import pallas as pl
from jax.experimental.pallas import tpu as pltpu

_HUFFMAN_PATHS = ((0, 1), (1, 0), (0, 0, 1), (1, 1, 0))
_NCOL = 16
_PACK = 8


def _body(h_ref, tw_ref, w0_ref, w1_ref, w2_ref, w3_ref, out_ref, wd_ref, tab_ref, wstk_ref):
    bm8 = h_ref.shape[0]
    hdim = h_ref.shape[1] // _PACK
    n = pl.num_programs(0) * bm8 * _PACK

    @pl.when(pl.program_id(0) == 0)
    def _():
        wd_ref[...] = jnp.zeros_like(wd_ref)
        wstk_ref[...] = jnp.zeros_like(wstk_ref)
        wstk_ref[0:2, :] = w0_ref[...]
        wstk_ref[2:4, :] = w1_ref[...]
        wstk_ref[4:7, :] = w2_ref[...]
        wstk_ref[7:10, :] = w3_ref[...]
        eye = (
            jax.lax.broadcasted_iota(jnp.int32, (_NCOL, _NCOL), 0)
            == jax.lax.broadcasted_iota(jnp.int32, (_NCOL, _NCOL), 1)
        ).astype(jnp.float32)
        wt = jax.lax.dot_general(
            wstk_ref[...], eye, (((0,), (0,)), ((), ())),
            preferred_element_type=jnp.float32,
        )
        for j in range(_PACK):
            wd_ref[j * hdim : (j + 1) * hdim, j * _NCOL : (j + 1) * _NCOL] = wt
        r8 = jax.lax.broadcasted_iota(jnp.int32, (8, 128), 0)
        c16 = jax.lax.broadcasted_iota(jnp.int32, (8, 128), 1) % _NCOL
        tab = jnp.zeros((8, 128), jnp.float32)
        off = 0
        for w, path in enumerate(_HUFFMAN_PATHS):
            lw = len(path)
            coeff = 1.0 / (lw * n)
            in_w = (c16 >= off) & (c16 < off + lw)
            tab = jnp.where((r8 == w) & in_w, coeff, tab)
            ones = [off + j2 for j2, bit in enumerate(path) if bit == 1]
            in_ones = (c16 >= ones[0]) & (c16 < ones[-1] + 1)
            tab = jnp.where((r8 == w + 4) & in_ones, coeff, tab)
            off += lw
        tab_ref[...] = tab
        out_ref[0, 0] = 0.0

    h2 = h_ref[...]
    x = jnp.dot(h2, wd_ref[...], preferred_element_type=jnp.float32)
    twf = tw_ref[...].astype(jnp.float32)
    rep = (
        jax.lax.broadcasted_iota(jnp.int32, (8, 128), 1) // _NCOL
        == jax.lax.broadcasted_iota(jnp.int32, (8, 128), 0)
    ).astype(jnp.float32)
    twp = jnp.dot(twf, rep, preferred_element_type=jnp.float32)
    soft = jnp.maximum(x, 0.0) + jnp.log1p(jnp.exp(-jnp.abs(x)))
    a = jnp.zeros_like(x)
    b = jnp.zeros_like(x)
    for w in range(4):
        m = twp == w
        a = a + jnp.where(m, jnp.broadcast_to(tab_ref[w : w + 1, :], x.shape), 0.0)
        b = b + jnp.where(
            m, jnp.broadcast_to(tab_ref[w + 4 : w + 5, :], x.shape), 0.0
        )
    out_ref[0, 0] += jnp.sum(a * soft) - jnp.sum(b * x)


@functools.partial(jax.jit, static_argnames=("interpret", "bm8"))
def kernel(hidden, target_words, W_0, W_1, W_2, W_3, interpret=False, bm8=256):
    batch, hdim = hidden.shape
    rows8 = batch // _PACK
    grid = rows8 // bm8
    h2 = hidden.reshape(rows8, _PACK * hdim)
    tw8 = target_words.astype(jnp.int32).reshape(rows8, _PACK)

    full = lambda shape: pl.BlockSpec(shape, lambda i: (0, 0))
    out = pl.pallas_call(
        _body,
        grid=(grid,),
        in_specs=[
            pl.BlockSpec((bm8, _PACK * hdim), lambda i: (i, 0)),
            pl.BlockSpec((bm8, _PACK), lambda i: (i, 0)),
            full(W_0.shape),
            full(W_1.shape),
            full(W_2.shape),
            full(W_3.shape),
        ],
        out_specs=pl.BlockSpec(
            (1, 1), lambda i: (0, 0), memory_space=pltpu.SMEM
        ),
        out_shape=jax.ShapeDtypeStruct((1, 1), jnp.float32),
        scratch_shapes=[
            pltpu.VMEM((_PACK * hdim, _PACK * _NCOL), jnp.float32),
            pltpu.VMEM((8, 128), jnp.float32),
            pltpu.VMEM((_NCOL, 512), jnp.float32),
        ],
        interpret=interpret,
    )(h2, tw8, W_0, W_1, W_2, W_3)
    return out[0, 0]


# speedup vs baseline: 1.8528x; 1.8528x over previous
import functools

import jax
import jax.numpy as jnp
from jax.experimental import pallas as pl
from jax.experimental.pallas import tpu as pltpu

_HUFFMAN_PATHS = ((0, 1), (1, 0), (0, 0, 1), (1, 1, 0))
_NCOL = 16


def _coeff_tables(n):
    r = jax.lax.broadcasted_iota(jnp.int32, (8, _NCOL), 0)
    c = jax.lax.broadcasted_iota(jnp.int32, (8, _NCOL), 1)
    a = jnp.zeros((8, _NCOL), jnp.float32)
    b = jnp.zeros((8, _NCOL), jnp.float32)
    off = 0
    for w, path in enumerate(_HUFFMAN_PATHS):
        lw = len(path)
        coeff = 1.0 / (lw * n)
        a = jnp.where((r == w) & (c >= off) & (c < off + lw), coeff, a)
        ones = [off + j for j, bit in enumerate(path) if bit == 1]
        b = jnp.where(
            (r == w) & (c >= ones[0]) & (c < ones[-1] + 1), coeff, b
        )
        off += lw
    return a, b


def _body(h_ref, tw_ref, w0_ref, w1_ref, w2_ref, w3_ref, out_ref, wt_ref, wstk_ref):
    bm = h_ref.shape[0]
    n = pl.num_programs(0) * bm

    @pl.when(pl.program_id(0) == 0)
    def _():
        wstk_ref[...] = jnp.zeros_like(wstk_ref)
        wstk_ref[0:2, :] = w0_ref[...]
        wstk_ref[2:4, :] = w1_ref[...]
        wstk_ref[4:7, :] = w2_ref[...]
        wstk_ref[7:10, :] = w3_ref[...]
        eye = (
            jax.lax.broadcasted_iota(jnp.int32, (_NCOL, _NCOL), 0)
            == jax.lax.broadcasted_iota(jnp.int32, (_NCOL, _NCOL), 1)
        ).astype(jnp.float32)
        wt_ref[...] = jax.lax.dot_general(
            wstk_ref[...], eye, (((0,), (0,)), ((), ())),
            preferred_element_type=jnp.float32,
        )
        out_ref[0, 0] = 0.0

    h = h_ref[...]
    tw = tw_ref[...]
    x = jnp.dot(h, wt_ref[...], preferred_element_type=jnp.float32)
    soft = jnp.maximum(x, 0.0) + jnp.log1p(jnp.exp(-jnp.abs(x)))
    onehot = (tw == jax.lax.broadcasted_iota(jnp.int32, (bm, 8), 1)).astype(
        jnp.float32
    )
    s_tab = jax.lax.dot_general(
        onehot, soft, (((0,), (0,)), ((), ())),
        preferred_element_type=jnp.float32,
    )
    x_tab = jax.lax.dot_general(
        onehot, x, (((0,), (0,)), ((), ())),
        preferred_element_type=jnp.float32,
    )
    a_tab, b_tab = _coeff_tables(n)
    out_ref[0, 0] += jnp.sum(a_tab * s_tab - b_tab * x_tab)


@functools.partial(jax.jit, static_argnames=("interpret", "bm"))
def kernel(hidden, target_words, W_0, W_1, W_2, W_3, interpret=False, bm=1024):
    batch, hdim = hidden.shape
    grid = batch // bm
    tw2d = target_words.astype(jnp.int32).reshape(batch, 1)

    full = lambda shape: pl.BlockSpec(shape, lambda i: (0, 0))
    out = pl.pallas_call(
        _body,
        grid=(grid,),
        in_specs=[
            pl.BlockSpec((bm, hdim), lambda i: (i, 0)),
            pl.BlockSpec((bm, 1), lambda i: (i, 0)),
            full(W_0.shape),
            full(W_1.shape),
            full(W_2.shape),
            full(W_3.shape),
        ],
        out_specs=pl.BlockSpec(
            (1, 1), lambda i: (0, 0), memory_space=pltpu.SMEM
        ),
        out_shape=jax.ShapeDtypeStruct((1, 1), jnp.float32),
        scratch_shapes=[
            pltpu.VMEM((hdim, _NCOL), jnp.float32),
            pltpu.VMEM((_NCOL, hdim), jnp.float32),
        ],
        interpret=interpret,
    )(hidden, tw2d, W_0, W_1, W_2, W_3)
    return out[0, 0]


# speedup vs baseline: 2.0304x; 1.0959x over previous
import functools

import jax
import jax.numpy as jnp
from jax.experimental import pallas as pl
from jax.experimental.pallas import tpu as pltpu

_HUFFMAN_PATHS = ((0, 1), (1, 0), (0, 0, 1), (1, 1, 0))
_NCOL = 16


def _coeff_tables(n):
    r = jax.lax.broadcasted_iota(jnp.int32, (8, _NCOL), 0)
    c = jax.lax.broadcasted_iota(jnp.int32, (8, _NCOL), 1)
    a = jnp.zeros((8, _NCOL), jnp.float32)
    b = jnp.zeros((8, _NCOL), jnp.float32)
    off = 0
    for w, path in enumerate(_HUFFMAN_PATHS):
        lw = len(path)
        coeff = 1.0 / (lw * n)
        a = jnp.where((r == w) & (c >= off) & (c < off + lw), coeff, a)
        ones = [off + j for j, bit in enumerate(path) if bit == 1]
        b = jnp.where(
            (r == w) & (c >= ones[0]) & (c < ones[-1] + 1), coeff, b
        )
        off += lw
    return a, b


def _body(h_ref, tw_ref, w0_ref, w1_ref, w2_ref, w3_ref, out_ref, wt_ref, wstk_ref):
    bm = h_ref.shape[0]
    n = pl.num_programs(0) * bm

    @pl.when(pl.program_id(0) == 0)
    def _():
        wstk_ref[...] = jnp.zeros_like(wstk_ref)
        wstk_ref[0:2, :] = w0_ref[...]
        wstk_ref[2:4, :] = w1_ref[...]
        wstk_ref[4:7, :] = w2_ref[...]
        wstk_ref[7:10, :] = w3_ref[...]
        eye = (
            jax.lax.broadcasted_iota(jnp.int32, (_NCOL, _NCOL), 0)
            == jax.lax.broadcasted_iota(jnp.int32, (_NCOL, _NCOL), 1)
        ).astype(jnp.float32)
        wt_ref[...] = jax.lax.dot_general(
            wstk_ref[...], eye, (((0,), (0,)), ((), ())),
            preferred_element_type=jnp.float32,
        )
        out_ref[0, 0] = 0.0

    h = h_ref[...]
    tw = tw_ref[...]
    x = jnp.dot(h, wt_ref[...], preferred_element_type=jnp.float32)
    soft = jnp.maximum(x, 0.0) + jnp.log1p(jnp.exp(-jnp.abs(x)))
    onehot = (tw == jax.lax.broadcasted_iota(jnp.int32, (bm, 8), 1)).astype(
        jnp.float32
    )
    s_tab = jax.lax.dot_general(
        onehot, soft, (((0,), (0,)), ((), ())),
        preferred_element_type=jnp.float32,
    )
    x_tab = jax.lax.dot_general(
        onehot, x, (((0,), (0,)), ((), ())),
        preferred_element_type=jnp.float32,
    )
    a_tab, b_tab = _coeff_tables(n)
    out_ref[0, 0] += jnp.sum(a_tab * s_tab - b_tab * x_tab)


@functools.partial(jax.jit, static_argnames=("interpret", "bm"))
def kernel(hidden, target_words, W_0, W_1, W_2, W_3, interpret=False, bm=2048):
    batch, hdim = hidden.shape
    grid = batch // bm
    tw2d = target_words.astype(jnp.int32).reshape(batch, 1)

    full = lambda shape: pl.BlockSpec(shape, lambda i: (0, 0))
    out = pl.pallas_call(
        _body,
        grid=(grid,),
        in_specs=[
            pl.BlockSpec((bm, hdim), lambda i: (i, 0)),
            pl.BlockSpec((bm, 1), lambda i: (i, 0)),
            full(W_0.shape),
            full(W_1.shape),
            full(W_2.shape),
            full(W_3.shape),
        ],
        out_specs=pl.BlockSpec(
            (1, 1), lambda i: (0, 0), memory_space=pltpu.SMEM
        ),
        out_shape=jax.ShapeDtypeStruct((1, 1), jnp.float32),
        scratch_shapes=[
            pltpu.VMEM((hdim, _NCOL), jnp.float32),
            pltpu.VMEM((_NCOL, hdim), jnp.float32),
        ],
        interpret=interpret,
    )(hidden, tw2d, W_0, W_1, W_2, W_3)
    return out[0, 0]
